# baseline (device time: 88325 ns/iter reference)
import jax
import jax.numpy as jnp
from jax import lax
from jax.experimental import pallas as pl
from jax.experimental.pallas import tpu as pltpu

NQ = 8

DX = 464
RZ = 280
RY = 280

CO = (5, 3, 4, 6, 7, 2, 0, 1)
RELAY_PIECES = {
    0: [(5, 640, 104, "z", 0), (5, 744, 24, "y", 3)],
    1: [(3, 464, 48, "z", 1)],
    2: [(4, 512, 128, "z", 2)],
    3: [(6, 768, 128, "y", 4)],
    4: [(7, 896, 128, "y", 5)],
}


def kernel(x):
    m, n = x.shape
    Q = m // 4
    C = Q // NQ

    def body(
        x_hbm, o_hbm, xv, remote, ov,
        p1_ssem, p1_rsem, p2y_ssem, p2y_rsem, p2z_ssem, p2z_rsem,
        rl_ssem, rl_rsem, cin_sem, cout_sem,
    ):
        my_x = lax.axis_index("x")
        my_y = lax.axis_index("y")
        my_z = lax.axis_index("z")
        xn = (1 - my_x, my_y, my_z)
        yn = (my_x, 1 - my_y, my_z)
        zn = (my_x, my_y, 1 - my_z)

        q = 2 * my_y + my_z
        q_y = 2 * (1 - my_y) + my_z
        q_z = 2 * my_y + (1 - my_z)
        q_d = 2 * (1 - my_y) + (1 - my_z)

        def lcopy(start, size, sem_idx):
            return pltpu.make_async_copy(
                x_hbm.at[pl.ds(start, size)],
                xv.at[pl.ds(start, size)],
                cin_sem.at[sem_idx],
            )

        lc_q = [lcopy(q * Q + c * C, C, c) for c in range(NQ)]
        lc_d = lcopy(q_d * Q, DX, NQ)
        lc_qy = lcopy(q_y * Q, Q, NQ + 1)
        lc_qz = lcopy(q_z * Q, Q, NQ + 2)
        lc_dt = lcopy(q_d * Q + DX, Q - DX, NQ + 3)
        for c in CO:
            lc_q[c].start()
        lc_d.start()
        lc_qy.start()
        lc_qz.start()
        lc_dt.start()

        barrier_sem = pltpu.get_barrier_semaphore()
        for nbr in (xn, yn, zn):
            pl.semaphore_signal(
                barrier_sem, inc=1, device_id=nbr,
                device_id_type=pl.DeviceIdType.MESH,
            )
        pl.semaphore_wait(barrier_sem, 3)

        def copy(src, dst, ssem, rsem, dev):
            return pltpu.make_async_remote_copy(
                src_ref=src, dst_ref=dst, send_sem=ssem, recv_sem=rsem,
                device_id=dev, device_id_type=pl.DeviceIdType.MESH,
            )

        def add_rows(start, size):
            ov[pl.ds(start, size), :] = (
                xv[pl.ds(start, size), :] + remote[pl.ds(start, size), :]
            )

        p1 = [
            copy(
                xv.at[pl.ds(q * Q + c * C, C)],
                remote.at[pl.ds(q * Q + c * C, C)],
                p1_ssem.at[c], p1_rsem.at[c], xn,
            )
            for c in range(NQ)
        ]
        p1d = copy(
            xv.at[pl.ds(q_d * Q, DX)],
            remote.at[pl.ds(q_d * Q, DX)],
            p1_ssem.at[NQ], p1_rsem.at[NQ], xn,
        )
        for c in CO:
            lc_q[c].wait()
            p1[c].start()
        lc_d.wait()
        p1d.start()

        p2y_in = [
            copy(
                remote.at[pl.ds(q_y * Q + c * C, C)],
                remote.at[pl.ds(q_y * Q + c * C, C)],
                p2y_ssem.at[c], p2y_rsem.at[c], yn,
            )
            for c in range(NQ)
        ]
        p2z_in = [
            copy(
                remote.at[pl.ds(q_z * Q + c * C, C)],
                remote.at[pl.ds(q_z * Q + c * C, C)],
                p2z_ssem.at[c], p2z_rsem.at[c], zn,
            )
            for c in range(NQ)
        ]

        rl_out, rl_in = [], []
        p2y_out, p2z_out = [], []

        def consume_position(j):
            cc = CO[j]
            if j == 0:
                lc_qy.wait()
                lc_qz.wait()
            p2y_in[cc].wait_recv()
            p2z_in[cc].wait_recv()
            for (sc, r0, nr, link, sem) in RELAY_PIECES.get(j, ()):
                if link == "z":
                    out = copy(
                        remote.at[pl.ds(q_y * Q + r0, nr)],
                        remote.at[pl.ds(q_y * Q + r0, nr)],
                        rl_ssem.at[sem], rl_rsem.at[sem], zn,
                    )
                else:
                    out = copy(
                        remote.at[pl.ds(q_z * Q + r0, nr)],
                        remote.at[pl.ds(q_z * Q + r0, nr)],
                        rl_ssem.at[sem], rl_rsem.at[sem], yn,
                    )
                out.start()
                rl_out.append(out)
                rl_in.append(
                    copy(
                        remote.at[pl.ds(q_d * Q + r0, nr)],
                        remote.at[pl.ds(q_d * Q + r0, nr)],
                        rl_ssem.at[sem], rl_rsem.at[sem],
                        zn if link == "z" else yn,
                    )
                )
            add_rows(q_y * Q + cc * C, C)
            add_rows(q_z * Q + cc * C, C)

        for idx, c in enumerate(CO):
            p1[c].wait_recv()
            src = remote.at[pl.ds(q * Q + c * C, C)]
            ry = copy(src, src, p2y_ssem.at[c], p2y_rsem.at[c], yn)
            rz = copy(src, src, p2z_ssem.at[c], p2z_rsem.at[c], zn)
            ry.start()
            rz.start()
            p2y_out.append(ry)
            p2z_out.append(rz)
            add_rows(q * Q + c * C, C)
            if idx >= 2:
                consume_position(idx - 2)

        def ocopy(start, size, sem_idx):
            return pltpu.make_async_copy(
                ov.at[pl.ds(start, size)],
                o_hbm.at[pl.ds(start, size)],
                cout_sem.at[sem_idx],
            )

        co_q = ocopy(q * Q, Q, 0)
        co_q.start()

        consume_position(NQ - 2)
        consume_position(NQ - 1)
        co_qy = ocopy(q_y * Q, Q, 1)
        co_qz = ocopy(q_z * Q, Q, 2)
        co_qy.start()
        co_qz.start()

        p1d.wait_recv()
        add_rows(q_d * Q, DX)
        for r in rl_in:
            r.wait_recv()
        lc_dt.wait()
        add_rows(q_d * Q + DX, RZ + RY)
        co_qd = ocopy(q_d * Q, Q, 3)
        co_qd.start()

        for r in p1:
            r.wait_send()
        p1d.wait_send()
        for r in p2y_out:
            r.wait_send()
        for r in p2z_out:
            r.wait_send()
        for r in rl_out:
            r.wait_send()
        co_q.wait()
        co_qy.wait()
        co_qz.wait()
        co_qd.wait()

    return pl.pallas_call(
        body,
        out_shape=jax.ShapeDtypeStruct((m, n), x.dtype),
        in_specs=[pl.BlockSpec(memory_space=pltpu.MemorySpace.HBM)],
        out_specs=pl.BlockSpec(memory_space=pltpu.MemorySpace.HBM),
        scratch_shapes=[
            pltpu.VMEM((m, n), x.dtype),
            pltpu.VMEM((m, n), x.dtype),
            pltpu.VMEM((m, n), x.dtype),
            pltpu.SemaphoreType.DMA((NQ + 1,)),
            pltpu.SemaphoreType.DMA((NQ + 1,)),
            pltpu.SemaphoreType.DMA((NQ,)),
            pltpu.SemaphoreType.DMA((NQ,)),
            pltpu.SemaphoreType.DMA((NQ,)),
            pltpu.SemaphoreType.DMA((NQ,)),
            pltpu.SemaphoreType.DMA((6,)),
            pltpu.SemaphoreType.DMA((6,)),
            pltpu.SemaphoreType.DMA((NQ + 4,)),
            pltpu.SemaphoreType.DMA((4,)),
        ],
        compiler_params=pltpu.CompilerParams(
            collective_id=0, vmem_limit_bytes=100 * 1024 * 1024,
        ),
    )(x)


# device time: 82883 ns/iter; 1.0657x vs baseline; 1.0657x over previous
import jax
import jax.numpy as jnp
from jax import lax
from jax.experimental import pallas as pl
from jax.experimental.pallas import tpu as pltpu

NQ = 8

DX = 464
RZ = 280
RY = 280

CO = (5, 3, 4, 6, 7, 2, 0, 1)
RELAY_PIECES = {
    0: [(5, 640, 104, "z", 0), (5, 744, 24, "y", 3)],
    1: [(3, 464, 48, "z", 1)],
    2: [(4, 512, 128, "z", 2)],
    3: [(6, 768, 128, "y", 4)],
    4: [(7, 896, 128, "y", 5)],
}


def kernel(x):
    m, n = x.shape
    Q = m // 4
    C = Q // NQ

    def body(
        x_ref, out_ref, remote,
        p1_ssem, p1_rsem, p2y_ssem, p2y_rsem, p2z_ssem, p2z_rsem,
        rl_ssem, rl_rsem,
    ):
        my_x = lax.axis_index("x")
        my_y = lax.axis_index("y")
        my_z = lax.axis_index("z")
        xn = (1 - my_x, my_y, my_z)
        yn = (my_x, 1 - my_y, my_z)
        zn = (my_x, my_y, 1 - my_z)

        q = 2 * my_y + my_z
        q_y = 2 * (1 - my_y) + my_z
        q_z = 2 * my_y + (1 - my_z)
        q_d = 2 * (1 - my_y) + (1 - my_z)

        barrier_sem = pltpu.get_barrier_semaphore()
        for nbr in (xn, yn, zn):
            pl.semaphore_signal(
                barrier_sem, inc=1, device_id=nbr,
                device_id_type=pl.DeviceIdType.MESH,
            )
        pl.semaphore_wait(barrier_sem, 3)

        def copy(src, dst, ssem, rsem, dev):
            return pltpu.make_async_remote_copy(
                src_ref=src, dst_ref=dst, send_sem=ssem, recv_sem=rsem,
                device_id=dev, device_id_type=pl.DeviceIdType.MESH,
            )

        def add_rows(start, size):
            out_ref[pl.ds(start, size), :] = (
                x_ref[pl.ds(start, size), :] + remote[pl.ds(start, size), :]
            )

        p1 = [
            copy(
                x_ref.at[pl.ds(q * Q + c * C, C)],
                remote.at[pl.ds(q * Q + c * C, C)],
                p1_ssem.at[c], p1_rsem.at[c], xn,
            )
            for c in range(NQ)
        ]
        p1d = copy(
            x_ref.at[pl.ds(q_d * Q, DX)],
            remote.at[pl.ds(q_d * Q, DX)],
            p1_ssem.at[NQ], p1_rsem.at[NQ], xn,
        )
        for c in CO:
            p1[c].start()
        p1d.start()

        p2y_in = [
            copy(
                remote.at[pl.ds(q_y * Q + c * C, C)],
                remote.at[pl.ds(q_y * Q + c * C, C)],
                p2y_ssem.at[c], p2y_rsem.at[c], yn,
            )
            for c in range(NQ)
        ]
        p2z_in = [
            copy(
                remote.at[pl.ds(q_z * Q + c * C, C)],
                remote.at[pl.ds(q_z * Q + c * C, C)],
                p2z_ssem.at[c], p2z_rsem.at[c], zn,
            )
            for c in range(NQ)
        ]

        rl_out, rl_in = [], []
        p2y_out, p2z_out = [], []

        def consume_position(j):
            cc = CO[j]
            p2y_in[cc].wait_recv()
            p2z_in[cc].wait_recv()
            for (sc, r0, nr, link, sem) in RELAY_PIECES.get(j, ()):
                if link == "z":
                    out = copy(
                        remote.at[pl.ds(q_y * Q + r0, nr)],
                        remote.at[pl.ds(q_y * Q + r0, nr)],
                        rl_ssem.at[sem], rl_rsem.at[sem], zn,
                    )
                else:
                    out = copy(
                        remote.at[pl.ds(q_z * Q + r0, nr)],
                        remote.at[pl.ds(q_z * Q + r0, nr)],
                        rl_ssem.at[sem], rl_rsem.at[sem], yn,
                    )
                out.start()
                rl_out.append(out)
                rl_in.append(
                    copy(
                        remote.at[pl.ds(q_d * Q + r0, nr)],
                        remote.at[pl.ds(q_d * Q + r0, nr)],
                        rl_ssem.at[sem], rl_rsem.at[sem],
                        zn if link == "z" else yn,
                    )
                )
            add_rows(q_y * Q + cc * C, C)
            add_rows(q_z * Q + cc * C, C)

        for idx, c in enumerate(CO):
            p1[c].wait_recv()
            src = remote.at[pl.ds(q * Q + c * C, C)]
            ry = copy(src, src, p2y_ssem.at[c], p2y_rsem.at[c], yn)
            rz = copy(src, src, p2z_ssem.at[c], p2z_rsem.at[c], zn)
            ry.start()
            rz.start()
            p2y_out.append(ry)
            p2z_out.append(rz)
            add_rows(q * Q + c * C, C)
            if idx >= 2:
                consume_position(idx - 2)

        consume_position(NQ - 2)
        consume_position(NQ - 1)

        p1d.wait_recv()
        add_rows(q_d * Q, DX)
        for r in rl_in:
            r.wait_recv()
        add_rows(q_d * Q + DX, RZ + RY)

        for r in p1:
            r.wait_send()
        p1d.wait_send()
        for r in p2y_out:
            r.wait_send()
        for r in p2z_out:
            r.wait_send()
        for r in rl_out:
            r.wait_send()

    return pl.pallas_call(
        body,
        out_shape=jax.ShapeDtypeStruct((m, n), x.dtype),
        in_specs=[pl.BlockSpec(memory_space=pltpu.VMEM)],
        out_specs=pl.BlockSpec(memory_space=pltpu.VMEM),
        scratch_shapes=[
            pltpu.VMEM((m, n), x.dtype),
            pltpu.SemaphoreType.DMA((NQ + 1,)),
            pltpu.SemaphoreType.DMA((NQ + 1,)),
            pltpu.SemaphoreType.DMA((NQ,)),
            pltpu.SemaphoreType.DMA((NQ,)),
            pltpu.SemaphoreType.DMA((NQ,)),
            pltpu.SemaphoreType.DMA((NQ,)),
            pltpu.SemaphoreType.DMA((6,)),
            pltpu.SemaphoreType.DMA((6,)),
        ],
        compiler_params=pltpu.CompilerParams(collective_id=0),
    )(x)
